# Initial kernel scaffold; baseline (speedup 1.0000x reference)
import jax
import jax.numpy as jnp
from jax.experimental import pallas as pl


def kernel(Cls, Shape, Offset):
    raise NotImplementedError("write your pallas kernel here")


# speedup vs baseline: 3.1165x; 3.1165x over previous
import functools

import jax
import jax.numpy as jnp
from jax.experimental import pallas as pl
from jax.experimental.pallas import tpu as pltpu

_TOPK = 60
_THRESHOLD = 0.15
_NMS_THRESHOLD = 0.05
_NMS_TOPK = 20
_STRIDE = 4.0
_D = 24
_N = _D * _D * _D
_ROWS = 108
_ROWS_PAD = 112
_LANES = 128
_BS = 16
_NEG_BIG = -1e30
_IDX_BIG = 1 << 30


def _body(logit_ref, shp_ref, off_ref, out_ref, scores_ref, nidx_ref):
    x = logit_ref[...]
    scores_ref[...] = 1.0 / (1.0 + jnp.exp(-x))
    row_i = jax.lax.broadcasted_iota(jnp.int32, (_BS, _ROWS_PAD, _LANES), 1)
    lane_i = jax.lax.broadcasted_iota(jnp.int32, (_BS, _ROWS_PAD, _LANES), 2)
    nidx_ref[...] = row_i * _LANES + lane_i

    lane64 = jax.lax.broadcasted_iota(jnp.int32, (_BS, 64), 1)

    def extract(it, carry):
        acc_s, acc_cz, acc_cy, acc_cx, acc_dz, acc_dy, acc_dx = carry
        s = scores_ref[...]
        nidx = nidx_ref[...]
        m = jnp.max(s, axis=(1, 2), keepdims=True)
        eq = s == m
        iw = jnp.where(eq, nidx, _IDX_BIG)
        im = jnp.min(iw, axis=(1, 2), keepdims=True)
        sel = nidx == im
        scores_ref[...] = jnp.where(sel, -1.0, s)

        selF = sel.astype(jnp.float32)

        def pick(arr):
            return jnp.sum(arr * selF, axis=(1, 2), keepdims=True)

        ovz = pick(off_ref[:, 0])
        ovy = pick(off_ref[:, 1])
        ovx = pick(off_ref[:, 2])
        shz = pick(shp_ref[:, 0])
        shy = pick(shp_ref[:, 1])
        shx = pick(shp_ref[:, 2])

        im2 = im.reshape(_BS, 1)
        z = im2 // (_D * _D)
        rem = im2 - z * (_D * _D)
        y = rem // _D
        xx = rem - y * _D
        zf = z.astype(jnp.float32)
        yf = y.astype(jnp.float32)
        xf = xx.astype(jnp.float32)

        oh = (lane64 == it).astype(jnp.float32)
        acc_s = acc_s + m.reshape(_BS, 1) * oh
        acc_cz = acc_cz + ((zf + ovz.reshape(_BS, 1)) * _STRIDE) * oh
        acc_cy = acc_cy + ((yf + ovy.reshape(_BS, 1)) * _STRIDE) * oh
        acc_cx = acc_cx + ((xf + ovx.reshape(_BS, 1)) * _STRIDE) * oh
        acc_dz = acc_dz + (2.0 * shz.reshape(_BS, 1)) * oh
        acc_dy = acc_dy + (2.0 * shy.reshape(_BS, 1)) * oh
        acc_dx = acc_dx + (2.0 * shx.reshape(_BS, 1)) * oh
        return acc_s, acc_cz, acc_cy, acc_cx, acc_dz, acc_dy, acc_dx

    zeros = jnp.zeros((_BS, 64), dtype=jnp.float32)
    accs = jax.lax.fori_loop(0, _TOPK, extract, (zeros,) * 7)
    acc_s, acc_cz, acc_cy, acc_cx, acc_dz, acc_dy, acc_dx = accs

    s_all = acc_s[:, 0:_TOPK]
    cz = acc_cz[:, 0:_TOPK]
    cy = acc_cy[:, 0:_TOPK]
    cx = acc_cx[:, 0:_TOPK]
    dz = acc_dz[:, 0:_TOPK]
    dy = acc_dy[:, 0:_TOPK]
    dx = acc_dx[:, 0:_TOPK]

    loz, hiz = cz - dz / 2.0, cz + dz / 2.0
    loy, hiy = cy - dy / 2.0, cy + dy / 2.0
    lox, hix = cx - dx / 2.0, cx + dx / 2.0
    vol = (jnp.maximum(dz, 0.0) * jnp.maximum(dy, 0.0)) * jnp.maximum(dx, 0.0)

    lane = jax.lax.broadcasted_iota(jnp.int32, (_BS, _TOPK), 1)
    sup = jnp.zeros((_BS, _TOPK), dtype=jnp.bool_)
    keep = jnp.zeros((_BS, _TOPK), dtype=jnp.bool_)
    cnt = jnp.zeros((_BS, 1), dtype=jnp.int32)

    for i in range(_TOPK):
        ci = slice(i, i + 1)
        valid_i = s_all[:, ci] > _THRESHOLD
        take = valid_i & jnp.logical_not(sup[:, ci]) & (cnt < _NMS_TOPK)
        cnt = cnt + take.astype(jnp.int32)
        do_sup = take & (cnt < _NMS_TOPK)

        iz = jnp.maximum(jnp.minimum(hiz, hiz[:, ci]) -
                         jnp.maximum(loz, loz[:, ci]), 0.0)
        iy = jnp.maximum(jnp.minimum(hiy, hiy[:, ci]) -
                         jnp.maximum(loy, loy[:, ci]), 0.0)
        ix = jnp.maximum(jnp.minimum(hix, hix[:, ci]) -
                         jnp.maximum(lox, lox[:, ci]), 0.0)
        inter = (iz * iy) * ix
        union = (vol[:, ci] + vol) - inter
        iou = jnp.where(union > 0.0,
                        inter / jnp.maximum(union, 1e-12), 0.0)

        is_i = lane == i
        keep = keep | (take & is_i)
        sup = sup | (do_sup & ((iou > _NMS_THRESHOLD) | is_i))

    keepI = keep.astype(jnp.int32)
    r_io = jax.lax.broadcasted_iota(jnp.int32, (_BS, _TOPK, _TOPK), 1)
    i_io = jax.lax.broadcasted_iota(jnp.int32, (_BS, _TOPK, _TOPK), 2)
    tri = (i_io <= r_io).astype(jnp.int32)
    kr = jnp.sum(tri * keepI[:, None, :], axis=2)
    rank = kr - 1
    oh = (keep[:, None, :] & (rank[:, None, :] == r_io)).astype(jnp.float32)

    def compact(v):
        return jnp.sum(oh * v[:, None, :], axis=2)

    row_valid = lane < cnt

    def fill(v):
        return jnp.where(row_valid, v, -1.0)

    out_ref[0] = jnp.where(row_valid, 1.0, -1.0)
    out_ref[1] = fill(compact(s_all))
    out_ref[2] = fill(compact(cz))
    out_ref[3] = fill(compact(cy))
    out_ref[4] = fill(compact(cx))
    out_ref[5] = fill(compact(dz))
    out_ref[6] = fill(compact(dy))
    out_ref[7] = fill(compact(dx))


@functools.partial(jax.jit, static_argnums=())
def kernel(Cls, Shape, Offset):
    bs = Cls.shape[0]
    logits = Cls.reshape(bs, _ROWS, _LANES)
    logits = jnp.pad(logits, ((0, 0), (0, _ROWS_PAD - _ROWS), (0, 0)),
                     constant_values=_NEG_BIG)
    shp = Shape.reshape(bs, 3, _ROWS, _LANES)
    shp = jnp.pad(shp, ((0, 0), (0, 0), (0, _ROWS_PAD - _ROWS), (0, 0)))
    off = Offset.reshape(bs, 3, _ROWS, _LANES)
    off = jnp.pad(off, ((0, 0), (0, 0), (0, _ROWS_PAD - _ROWS), (0, 0)))

    out = pl.pallas_call(
        _body,
        out_shape=jax.ShapeDtypeStruct((8, _BS, _TOPK), jnp.float32),
        scratch_shapes=[
            pltpu.VMEM((_BS, _ROWS_PAD, _LANES), jnp.float32),
            pltpu.VMEM((_BS, _ROWS_PAD, _LANES), jnp.int32),
        ],
    )(logits, shp, off)
    return jnp.transpose(out, (1, 2, 0))


# speedup vs baseline: 3.2674x; 1.0484x over previous
import functools

import jax
import jax.numpy as jnp
from jax.experimental import pallas as pl
from jax.experimental.pallas import tpu as pltpu

_TOPK = 60
_THRESHOLD = 0.15
_NMS_THRESHOLD = 0.05
_NMS_TOPK = 20
_STRIDE = 4.0
_D = 24
_N = _D * _D * _D
_ROWS = 108
_ROWS_PAD = 112
_LANES = 128
_BS = 16
_NEG_BIG = -1e30
_IDX_BIG = 1 << 30


def _body(logit_ref, shp_ref, off_ref, out_ref, scores_ref, nidx_ref):
    x = logit_ref[...]
    scores_ref[...] = 1.0 / (1.0 + jnp.exp(-x))
    row_i = jax.lax.broadcasted_iota(jnp.int32, (_BS, _ROWS_PAD, _LANES), 1)
    lane_i = jax.lax.broadcasted_iota(jnp.int32, (_BS, _ROWS_PAD, _LANES), 2)
    nidx_ref[...] = row_i * _LANES + lane_i

    lane64 = jax.lax.broadcasted_iota(jnp.int32, (_BS, 64), 1)

    def extract(it, carry):
        acc_s, acc_n = carry
        s = scores_ref[...]
        nidx = nidx_ref[...]
        m = jnp.max(s, axis=(1, 2), keepdims=True)
        eq = s == m
        iw = jnp.where(eq, nidx, _IDX_BIG)
        im = jnp.min(iw, axis=(1, 2), keepdims=True)
        scores_ref[...] = jnp.where(nidx == im, -1.0, s)

        oh = lane64 == it
        acc_s = acc_s + jnp.where(oh, m.reshape(_BS, 1), 0.0)
        acc_n = acc_n + jnp.where(oh, im.reshape(_BS, 1), 0)
        return acc_s, acc_n

    acc_s, acc_n = jax.lax.fori_loop(
        0, _TOPK, extract,
        (jnp.zeros((_BS, 64), jnp.float32), jnp.zeros((_BS, 64), jnp.int32)))

    r = acc_n // _LANES
    l = acc_n - r * _LANES
    z = acc_n // (_D * _D)
    rem = acc_n - z * (_D * _D)
    y = rem // _D
    xx = rem - y * _D

    ohr = (jax.lax.broadcasted_iota(jnp.int32, (_BS, 64, _ROWS_PAD), 2)
           == r[:, :, None]).astype(jnp.float32)
    big = jnp.concatenate(
        [off_ref[:, 0], off_ref[:, 1], off_ref[:, 2],
         shp_ref[:, 0], shp_ref[:, 1], shp_ref[:, 2]], axis=2)
    rowdata = jax.lax.dot_general(
        ohr, big, (((2,), (1,)), ((0,), (0,))),
        precision=jax.lax.Precision.HIGHEST,
        preferred_element_type=jnp.float32)
    ohl = (jax.lax.broadcasted_iota(jnp.int32, (_BS, 64, _LANES), 2)
           == l[:, :, None]).astype(jnp.float32)

    def pick(c):
        return jnp.sum(rowdata[:, :, c * _LANES:(c + 1) * _LANES] * ohl,
                       axis=2)

    ovz, ovy, ovx = pick(0), pick(1), pick(2)
    shz, shy, shx = pick(3), pick(4), pick(5)
    acc_cz = (z.astype(jnp.float32) + ovz) * _STRIDE
    acc_cy = (y.astype(jnp.float32) + ovy) * _STRIDE
    acc_cx = (xx.astype(jnp.float32) + ovx) * _STRIDE
    acc_dz = 2.0 * shz
    acc_dy = 2.0 * shy
    acc_dx = 2.0 * shx

    s_all = acc_s[:, 0:_TOPK]
    cz = acc_cz[:, 0:_TOPK]
    cy = acc_cy[:, 0:_TOPK]
    cx = acc_cx[:, 0:_TOPK]
    dz = acc_dz[:, 0:_TOPK]
    dy = acc_dy[:, 0:_TOPK]
    dx = acc_dx[:, 0:_TOPK]

    loz, hiz = cz - dz / 2.0, cz + dz / 2.0
    loy, hiy = cy - dy / 2.0, cy + dy / 2.0
    lox, hix = cx - dx / 2.0, cx + dx / 2.0
    vol = (jnp.maximum(dz, 0.0) * jnp.maximum(dy, 0.0)) * jnp.maximum(dx, 0.0)

    lane = jax.lax.broadcasted_iota(jnp.int32, (_BS, _TOPK), 1)
    sup = jnp.zeros((_BS, _TOPK), dtype=jnp.bool_)
    keep = jnp.zeros((_BS, _TOPK), dtype=jnp.bool_)
    cnt = jnp.zeros((_BS, 1), dtype=jnp.int32)

    for i in range(_TOPK):
        ci = slice(i, i + 1)
        valid_i = s_all[:, ci] > _THRESHOLD
        take = valid_i & jnp.logical_not(sup[:, ci]) & (cnt < _NMS_TOPK)
        cnt = cnt + take.astype(jnp.int32)
        do_sup = take & (cnt < _NMS_TOPK)

        iz = jnp.maximum(jnp.minimum(hiz, hiz[:, ci]) -
                         jnp.maximum(loz, loz[:, ci]), 0.0)
        iy = jnp.maximum(jnp.minimum(hiy, hiy[:, ci]) -
                         jnp.maximum(loy, loy[:, ci]), 0.0)
        ix = jnp.maximum(jnp.minimum(hix, hix[:, ci]) -
                         jnp.maximum(lox, lox[:, ci]), 0.0)
        inter = (iz * iy) * ix
        union = (vol[:, ci] + vol) - inter
        iou = jnp.where(union > 0.0,
                        inter / jnp.maximum(union, 1e-12), 0.0)

        is_i = lane == i
        keep = keep | (take & is_i)
        sup = sup | (do_sup & ((iou > _NMS_THRESHOLD) | is_i))

    keepI = keep.astype(jnp.int32)
    r_io = jax.lax.broadcasted_iota(jnp.int32, (_BS, _TOPK, _TOPK), 1)
    i_io = jax.lax.broadcasted_iota(jnp.int32, (_BS, _TOPK, _TOPK), 2)
    tri = (i_io <= r_io).astype(jnp.int32)
    kr = jnp.sum(tri * keepI[:, None, :], axis=2)
    rank = kr - 1
    oh = (keep[:, None, :] & (rank[:, None, :] == r_io)).astype(jnp.float32)

    def compact(v):
        return jnp.sum(oh * v[:, None, :], axis=2)

    row_valid = lane < cnt

    def fill(v):
        return jnp.where(row_valid, v, -1.0)

    out_ref[0] = jnp.where(row_valid, 1.0, -1.0)
    out_ref[1] = fill(compact(s_all))
    out_ref[2] = fill(compact(cz))
    out_ref[3] = fill(compact(cy))
    out_ref[4] = fill(compact(cx))
    out_ref[5] = fill(compact(dz))
    out_ref[6] = fill(compact(dy))
    out_ref[7] = fill(compact(dx))


@functools.partial(jax.jit, static_argnums=())
def kernel(Cls, Shape, Offset):
    bs = Cls.shape[0]
    logits = Cls.reshape(bs, _ROWS, _LANES)
    logits = jnp.pad(logits, ((0, 0), (0, _ROWS_PAD - _ROWS), (0, 0)),
                     constant_values=_NEG_BIG)
    shp = Shape.reshape(bs, 3, _ROWS, _LANES)
    shp = jnp.pad(shp, ((0, 0), (0, 0), (0, _ROWS_PAD - _ROWS), (0, 0)))
    off = Offset.reshape(bs, 3, _ROWS, _LANES)
    off = jnp.pad(off, ((0, 0), (0, 0), (0, _ROWS_PAD - _ROWS), (0, 0)))

    out = pl.pallas_call(
        _body,
        out_shape=jax.ShapeDtypeStruct((8, _BS, _TOPK), jnp.float32),
        scratch_shapes=[
            pltpu.VMEM((_BS, _ROWS_PAD, _LANES), jnp.float32),
            pltpu.VMEM((_BS, _ROWS_PAD, _LANES), jnp.int32),
        ],
    )(logits, shp, off)
    return jnp.transpose(out, (1, 2, 0))
